# Initial kernel scaffold; baseline (speedup 1.0000x reference)
import jax
import jax.numpy as jnp
from jax.experimental import pallas as pl


def kernel(hidden_states, codebooks):
    raise NotImplementedError("write your pallas kernel here")


# speedup vs baseline: 17.3584x; 17.3584x over previous
import jax
import jax.numpy as jnp
from jax import lax
from jax.experimental import pallas as pl

N_TOKENS = 1024
DIM = 256
N_Q = 4
BINS = 512

BLOCK_T = 256


def _rvq_kernel(h_ref, cb_ref, codes_ref, quant_ref):
    residual = h_ref[:]
    quant = jnp.zeros_like(residual)
    ones8 = jnp.ones((8, DIM), jnp.float32)
    idx_cols = []
    for i in range(N_Q):
        cb = cb_ref[i]
        cnorm8 = lax.dot_general(
            ones8, cb * cb,
            dimension_numbers=(((1,), (1,)), ((), ())),
            preferred_element_type=jnp.float32,
            precision=lax.Precision.HIGHEST,
        )
        dots = lax.dot_general(
            residual, cb,
            dimension_numbers=(((1,), (1,)), ((), ())),
            preferred_element_type=jnp.float32,
            precision=lax.Precision.HIGHEST,
        )
        scores = cnorm8[0:1, :] - 2.0 * dots
        mins = jnp.min(scores, axis=1, keepdims=True)
        iota = lax.broadcasted_iota(jnp.int32, scores.shape, 1)
        idx2d = jnp.min(jnp.where(scores == mins, iota, BINS),
                        axis=1, keepdims=True)
        onehot = (iota == idx2d).astype(jnp.float32)
        chosen = lax.dot_general(
            onehot, cb,
            dimension_numbers=(((1,), (0,)), ((), ())),
            preferred_element_type=jnp.float32,
            precision=lax.Precision.HIGHEST,
        )
        quant = quant + chosen
        residual = residual - chosen
        idx_cols.append(idx2d)
    codes_ref[:] = jnp.concatenate(idx_cols, axis=1)
    quant_ref[:] = quant


def kernel(hidden_states, codebooks):
    grid = (N_TOKENS // BLOCK_T,)
    codes_t, quant = pl.pallas_call(
        _rvq_kernel,
        grid=grid,
        in_specs=[
            pl.BlockSpec((BLOCK_T, DIM), lambda j: (j, 0)),
            pl.BlockSpec((N_Q, BINS, DIM), lambda j: (0, 0, 0)),
        ],
        out_specs=[
            pl.BlockSpec((BLOCK_T, N_Q), lambda j: (j, 0)),
            pl.BlockSpec((BLOCK_T, DIM), lambda j: (j, 0)),
        ],
        out_shape=[
            jax.ShapeDtypeStruct((N_TOKENS, N_Q), jnp.int32),
            jax.ShapeDtypeStruct((N_TOKENS, DIM), jnp.float32),
        ],
    )(hidden_states, codebooks)
    return jnp.transpose(codes_t), quant


# speedup vs baseline: 25.6339x; 1.4767x over previous
import jax
import jax.numpy as jnp
from jax import lax
from jax.experimental import pallas as pl

N_TOKENS = 1024
DIM = 256
N_Q = 4
BINS = 512


def _split4(x):
    parts = []
    r = x
    for _ in range(4):
        c = r.astype(jnp.bfloat16).astype(jnp.float32)
        parts.append(c)
        r = r - c
    return parts


def _rvq_kernel(h_ref, cb_ref, codes_ref, quant_ref):
    residual = h_ref[:]
    ones8 = jnp.ones((8, DIM), jnp.float32)
    idx_cols = []
    for i in range(N_Q):
        cb = cb_ref[i]
        cnorm8 = lax.dot_general(
            ones8, cb * cb,
            dimension_numbers=(((1,), (1,)), ((), ())),
            preferred_element_type=jnp.float32,
            precision=lax.Precision.HIGHEST,
        )
        dots = lax.dot_general(
            residual, cb,
            dimension_numbers=(((1,), (1,)), ((), ())),
            preferred_element_type=jnp.float32,
            precision=lax.Precision.HIGHEST,
        )
        scores = cnorm8[0:1, :] - 2.0 * dots
        mins = jnp.min(scores, axis=1, keepdims=True)
        iota = lax.broadcasted_iota(jnp.int32, scores.shape, 1)
        idx2d = jnp.min(jnp.where(scores == mins, iota, BINS),
                        axis=1, keepdims=True)
        onehot = (iota == idx2d).astype(jnp.float32)
        chosen = jnp.zeros_like(residual)
        for part in _split4(cb):
            chosen = chosen + lax.dot_general(
                onehot, part,
                dimension_numbers=(((1,), (0,)), ((), ())),
                preferred_element_type=jnp.float32,
            )
        residual = residual - chosen
        idx_cols.append(idx2d)
    codes_ref[:] = jnp.concatenate(idx_cols, axis=1)
    quant_ref[:] = h_ref[:] - residual


def kernel(hidden_states, codebooks):
    codes_t, quant = pl.pallas_call(
        _rvq_kernel,
        out_shape=[
            jax.ShapeDtypeStruct((N_TOKENS, N_Q), jnp.int32),
            jax.ShapeDtypeStruct((N_TOKENS, DIM), jnp.float32),
        ],
    )(hidden_states, codebooks)
    return jnp.transpose(codes_t), quant


# speedup vs baseline: 27.3589x; 1.0673x over previous
import jax
import jax.numpy as jnp
from jax import lax
from jax.experimental import pallas as pl

N_TOKENS = 1024
DIM = 256
N_Q = 4
BINS = 512


def _split3(x):
    parts = []
    r = x
    for _ in range(3):
        c = r.astype(jnp.bfloat16).astype(jnp.float32)
        parts.append(c)
        r = r - c
    return parts


def _rvq_kernel(h_ref, cb_ref, codes_ref, quant_ref):
    residual = h_ref[:]
    ones8 = jnp.ones((8, DIM), jnp.float32)
    cb_all = cb_ref[:].reshape(N_Q * BINS, DIM)
    cnorm8 = lax.dot_general(
        ones8, cb_all * cb_all,
        dimension_numbers=(((1,), (1,)), ((), ())),
        preferred_element_type=jnp.float32,
        precision=lax.Precision.HIGHEST,
    )
    idx_cols = []
    for i in range(N_Q):
        cb = cb_ref[i]
        dots = lax.dot_general(
            residual, cb,
            dimension_numbers=(((1,), (1,)), ((), ())),
            preferred_element_type=jnp.float32,
            precision=lax.Precision.HIGHEST,
        )
        scores = cnorm8[0:1, i * BINS:(i + 1) * BINS] - 2.0 * dots
        mins = jnp.min(scores, axis=1, keepdims=True)
        iota = lax.broadcasted_iota(jnp.int32, scores.shape, 1)
        idx2d = jnp.min(jnp.where(scores == mins, iota, BINS),
                        axis=1, keepdims=True)
        onehot = (iota == idx2d).astype(jnp.float32)
        chosen = jnp.zeros_like(residual)
        for part in _split3(cb):
            chosen = chosen + lax.dot_general(
                onehot, part,
                dimension_numbers=(((1,), (0,)), ((), ())),
                preferred_element_type=jnp.float32,
            )
        residual = residual - chosen
        idx_cols.append(idx2d)
    codes_ref[:] = jnp.concatenate(idx_cols, axis=1)
    quant_ref[:] = h_ref[:] - residual


def kernel(hidden_states, codebooks):
    codes_t, quant = pl.pallas_call(
        _rvq_kernel,
        out_shape=[
            jax.ShapeDtypeStruct((N_TOKENS, N_Q), jnp.int32),
            jax.ShapeDtypeStruct((N_TOKENS, DIM), jnp.float32),
        ],
    )(hidden_states, codebooks)
    return jnp.transpose(codes_t), quant
